# baseline (speedup 1.0000x reference)
import functools

import jax
import jax.numpy as jnp
from jax import lax
from jax.experimental import pallas as pl
from jax.experimental.pallas import tpu as pltpu
from jax.experimental.pallas import tpu_sc as plsc

NW = 32
BB = 128
NROW = 3
NTR = 2
TP = 129


def _gather_kernel(L, d, B, table_hbm, idx_hbm, out_hbm,
                   idx_v, rows_v, tr_v, gsem, wsem):
    wid = lax.axis_index("s") * 2 + lax.axis_index("c")
    b0 = wid * BB
    pltpu.sync_copy(idx_hbm.at[wid], idx_v)
    lanes = lax.iota(jnp.int32, 16)

    def gather_copy(l, p):
        return pltpu.make_async_copy(
            table_hbm.at[idx_v.at[l]], rows_v.at[p], gsem)

    def write_copy(l, q):
        src = tr_v.at[q, :, pl.ds(0, BB)]
        dst = out_hbm.at[pl.ds(l * d, d), pl.ds(b0, BB)]
        return pltpu.make_async_copy(src, dst, wsem)

    gather_copy(0, 0).start()
    gather_copy(1, 1).start()

    @pl.loop(0, L)
    def _(l):
        p = lax.rem(l, NROW)
        q = lax.rem(l, NTR)
        gather_copy(l, p).wait()

        @pl.when(l >= NTR)
        def _():
            write_copy(l - NTR, q).wait()

        qs = jnp.full((16,), q, jnp.int32)

        @pl.loop(0, BB)
        def _(b):
            bs = jnp.full((16,), b, jnp.int32)
            for j in range(d // 16):
                vec = rows_v[p, b, pl.ds(j * 16, 16)]
                plsc.store_scatter(tr_v, [qs, j * 16 + lanes, bs], vec)

        write_copy(l, q).start()

        @pl.when(l + 2 < L)
        def _():
            gather_copy(l + 2, lax.rem(l + 2, NROW)).start()

    write_copy(L - 2, lax.rem(L - 2, NTR)).wait()
    write_copy(L - 1, lax.rem(L - 1, NTR)).wait()


def kernel(characters, tokens, table):
    B, L = tokens.shape
    V, D = table.shape

    idx = tokens.T.reshape(L, NW, BB).transpose(1, 0, 2).astype(jnp.int32)

    mesh = plsc.VectorSubcoreMesh(core_axis_name="c", subcore_axis_name="s")
    run = functools.partial(
        pl.kernel,
        out_type=jax.ShapeDtypeStruct((L * D, B), jnp.float32),
        mesh=mesh,
        compiler_params=pltpu.CompilerParams(
            use_tc_tiling_on_sc=False, needs_layout_passes=False),
        scratch_types=[
            pltpu.VMEM((L, BB), jnp.int32),
            pltpu.VMEM((NROW, BB, D), jnp.float32),
            pltpu.VMEM((NTR, D, TP), jnp.float32),
            pltpu.SemaphoreType.DMA,
            pltpu.SemaphoreType.DMA,
        ],
    )(functools.partial(_gather_kernel, L, D, B))

    out = run(table, idx)
    return out.reshape(L, D, B).transpose(2, 0, 1)


# speedup vs baseline: 1.0322x; 1.0322x over previous
import functools

import jax
import jax.numpy as jnp
from jax import lax
from jax.experimental import pallas as pl
from jax.experimental.pallas import tpu as pltpu
from jax.experimental.pallas import tpu_sc as plsc

NW = 32
BB = 128
NROW = 3
NTR = 2
TP = 129


def _gather_kernel(L, d, B, table_hbm, idx_hbm, out_hbm,
                   idx_v, rows_v, tr_v, gsem, wsem):
    wid = lax.axis_index("s") * 2 + lax.axis_index("c")
    b0 = wid * BB
    pltpu.sync_copy(idx_hbm.at[wid], idx_v)
    lanes = lax.iota(jnp.int32, 16)

    def gather_copy(l, p):
        return pltpu.make_async_copy(
            table_hbm.at[idx_v.at[l]], rows_v.at[p], gsem)

    def write_copy(l, q):
        src = tr_v.at[q, :, pl.ds(0, BB)]
        dst = out_hbm.at[pl.ds(l * d, d), pl.ds(b0, BB)]
        return pltpu.make_async_copy(src, dst, wsem)

    gather_copy(0, 0).start()
    gather_copy(1, 1).start()

    @pl.loop(0, L)
    def _(l):
        p = lax.rem(l, NROW)
        q = lax.rem(l, NTR)
        gather_copy(l, p).wait()

        @pl.when(l >= NTR)
        def _():
            write_copy(l - NTR, q).wait()

        qs = jnp.full((16,), q, jnp.int32)

        @pl.loop(0, BB // 16)
        def _(bg):
            for k in range(16):
                b = bg * 16 + k
                bs = jnp.full((16,), b, jnp.int32)
                for j in range(d // 16):
                    vec = rows_v[p, b, pl.ds(j * 16, 16)]
                    plsc.store_scatter(tr_v, [qs, j * 16 + lanes, bs], vec)

        write_copy(l, q).start()

        @pl.when(l + 2 < L)
        def _():
            gather_copy(l + 2, lax.rem(l + 2, NROW)).start()

    write_copy(L - 2, lax.rem(L - 2, NTR)).wait()
    write_copy(L - 1, lax.rem(L - 1, NTR)).wait()


def kernel(characters, tokens, table):
    B, L = tokens.shape
    V, D = table.shape

    idx = tokens.T.reshape(L, NW, BB).transpose(1, 0, 2).astype(jnp.int32)

    mesh = plsc.VectorSubcoreMesh(core_axis_name="c", subcore_axis_name="s")
    run = functools.partial(
        pl.kernel,
        out_type=jax.ShapeDtypeStruct((L * D, B), jnp.float32),
        mesh=mesh,
        compiler_params=pltpu.CompilerParams(
            use_tc_tiling_on_sc=False, needs_layout_passes=False),
        scratch_types=[
            pltpu.VMEM((L, BB), jnp.int32),
            pltpu.VMEM((NROW, BB, D), jnp.float32),
            pltpu.VMEM((NTR, D, TP), jnp.float32),
            pltpu.SemaphoreType.DMA,
            pltpu.SemaphoreType.DMA,
        ],
    )(functools.partial(_gather_kernel, L, D, B))

    out = run(table, idx)
    return out.reshape(L, D, B).transpose(2, 0, 1)


# speedup vs baseline: 1.0695x; 1.0361x over previous
import functools

import jax
import jax.numpy as jnp
from jax import lax
from jax.experimental import pallas as pl
from jax.experimental.pallas import tpu as pltpu
from jax.experimental.pallas import tpu_sc as plsc

NW = 32
CHUNK = 128
K = 2
NSET = 5


def _gather_kernel(n_chunks, table_hbm, idx_hbm, out_hbm,
                   idx_v, rows_v, gsem, wsem):
    n_groups = n_chunks // K
    wid = lax.axis_index("s") * 2 + lax.axis_index("c")
    base = wid * (n_chunks * CHUNK)
    pltpu.sync_copy(idx_hbm.at[wid], idx_v)

    def gather_copy(g, p, b):
        return pltpu.make_async_copy(
            table_hbm.at[idx_v.at[g * K + b]], rows_v.at[p, b], gsem)

    def write_copy(g, p, b):
        dst = out_hbm.at[pl.ds(base + (g * K + b) * CHUNK, CHUNK)]
        return pltpu.make_async_copy(rows_v.at[p, b], dst, wsem)

    def start_gathers(g, p):
        for b in range(K):
            gather_copy(g, p, b).start()

    start_gathers(0, 0)
    start_gathers(1, 1)

    @pl.loop(0, n_groups, step=NSET)
    def _(g0):
        for p in range(NSET):
            g = g0 + p
            for b in range(K):
                gather_copy(g, p, b).wait()
            for b in range(K):
                write_copy(g, p, b).start()
            pw = (p + 2) % NSET

            @pl.when(g >= 3)
            def _():
                for b in range(K):
                    write_copy(g - 3, pw, b).wait()

            @pl.when(g + 2 < n_groups)
            def _():
                start_gathers(g + 2, pw)

    for g in (n_groups - 3, n_groups - 2, n_groups - 1):
        for b in range(K):
            write_copy(g, g % NSET, b).wait()


def kernel(characters, tokens, table):
    B, L = tokens.shape
    V, D = table.shape
    N = B * L
    n_per_w = N // NW
    n_chunks = n_per_w // CHUNK

    idx = tokens.reshape(NW, n_chunks, CHUNK).astype(jnp.int32)

    mesh = plsc.VectorSubcoreMesh(core_axis_name="c", subcore_axis_name="s")
    run = functools.partial(
        pl.kernel,
        out_type=jax.ShapeDtypeStruct((N, D), jnp.float32),
        mesh=mesh,
        compiler_params=pltpu.CompilerParams(use_tc_tiling_on_sc=False),
        scratch_types=[
            pltpu.VMEM((n_chunks, CHUNK), jnp.int32),
            pltpu.VMEM((NSET, K, CHUNK, D), jnp.float32),
            pltpu.SemaphoreType.DMA,
            pltpu.SemaphoreType.DMA,
        ],
    )(functools.partial(_gather_kernel, n_chunks))

    out = run(table, idx)
    return out.reshape(B, L, D)
